# Initial kernel scaffold; baseline (speedup 1.0000x reference)
import jax
import jax.numpy as jnp
from jax.experimental import pallas as pl


def kernel(x_individuals, x_occupation, x_residence, edge_index_family, edge_index_occupation, edge_index_residence, population, edge_attributes, Wl_dir_occ, bl_dir_occ, Wr_dir_occ, Wl_dir_res, bl_dir_res, Wr_dir_res, Wl_msg, bl_msg, Wr_msg, Wl_inv_occ, bl_inv_occ, Wr_inv_occ, Wl_inv_res, bl_inv_res, Wr_inv_res, Wl_inv_ind, bl_inv_ind, Wr_inv_ind, P_occ, P_res, W_aggr, b_aggr):
    raise NotImplementedError("write your pallas kernel here")


# speedup vs baseline: 5.2428x; 5.2428x over previous
import functools

import jax
import jax.numpy as jnp
from jax import lax
from jax.experimental import pallas as pl
from jax.experimental.pallas import tpu as pltpu
from jax.experimental.pallas import tpu_sc as plsc

D = 128
HW = 80
N_IND = 10000
N_ATT = 1000
OFF_OCC = 10240
OFF_RES = 11264
NROWS = 12288
DUMMY = 12280
NC, NS = 2, 16
CH = 80
E_TOT = 320000 + 10000 + 10000
E_PT = 21280
NCHUNK = E_PT // CH
E_PAD = E_PT * NS
ROWS_PT = NROWS // NS


def _segsum_body(tables, srcs, dsts, zeros96, sums_out,
                 src_v, dst_v, rows_v, zero_v, sg0, sg1, ss0, ss1, acc):
    cid = lax.axis_index("c")
    sid = lax.axis_index("s")
    sg = (sg0, sg1)
    ss = (ss0, ss1)
    table = tables.at[cid]

    pltpu.sync_copy(zeros96, zero_v)
    pltpu.sync_copy(srcs.at[sid], src_v)
    pltpu.sync_copy(dsts.at[sid], dst_v)
    for t in range(ROWS_PT // 96):
        pltpu.sync_copy(zero_v, acc.at[pl.ds(sid * ROWS_PT + t * 96, 96)])
    plsc.subcore_barrier()

    def start_g(j, b):
        pltpu.async_copy(table.at[src_v.at[j]], rows_v.at[b], sg[b])

    def wait_g(b):
        pltpu.make_async_copy(table.at[src_v.at[0]], rows_v.at[b], sg[b]).wait()

    def start_s(j, b):
        pltpu.async_copy(rows_v.at[b], acc.at[dst_v.at[j]], ss[b], add=True)

    def wait_s(b):
        pltpu.make_async_copy(rows_v.at[b], acc.at[dst_v.at[0]], ss[b]).wait()

    start_g(0, 0)
    start_g(1, 1)

    @pl.loop(0, NCHUNK, step=2)
    def _(j0):
        for b in (0, 1):
            j = j0 + b
            wait_g(b)
            start_s(j, b)

            @pl.when(j + 2 < NCHUNK)
            def _():
                wait_s(b)
                start_g(j + 2, b)

    wait_s(0)
    wait_s(1)

    plsc.subcore_barrier()
    pltpu.sync_copy(acc.at[pl.ds(sid * ROWS_PT, ROWS_PT)],
                    sums_out.at[cid, pl.ds(sid * ROWS_PT, ROWS_PT)])


def _segsum_sc(tables, srcs, dsts, zeros96, interpret=False):
    return pl.kernel(
        _segsum_body,
        out_type=jax.ShapeDtypeStruct((NC, NROWS, HW), jnp.float32),
        mesh=plsc.VectorSubcoreMesh(core_axis_name="c", subcore_axis_name="s",
                                    num_cores=NC, num_subcores=NS),
        scratch_types=[
            pltpu.VMEM((NCHUNK, CH), jnp.int32),
            pltpu.VMEM((NCHUNK, CH), jnp.int32),
            pltpu.VMEM((2, CH, HW), jnp.float32),
            pltpu.VMEM((96, HW), jnp.float32),
            pltpu.SemaphoreType.DMA,
            pltpu.SemaphoreType.DMA,
            pltpu.SemaphoreType.DMA,
            pltpu.SemaphoreType.DMA,
            pltpu.VMEM_SHARED((NROWS, HW), jnp.float32),
        ],
        compiler_params=pltpu.CompilerParams(use_tc_tiling_on_sc=False),
        interpret=interpret,
    )(tables, srcs, dsts, zeros96)


def _dense_layer(sums, xs, Wl3, Wr3, bl3, relu, interpret=False):
    nblk = NROWS // 1024

    def body(s_ref, x_ref, wl_ref, wr_ref, bl_ref, o_ref):
        s0 = s_ref[0]
        s1 = s_ref[1]
        inv = 1.0 / jnp.maximum(s0[:, 64:65], 1.0)
        wl = wl_ref[0]
        out = jnp.dot(s0[:, :64] * inv, wl[:64],
                      preferred_element_type=jnp.float32)
        out += jnp.dot(s1[:, :64] * inv, wl[64:],
                       preferred_element_type=jnp.float32)
        out += bl_ref[0]
        out += jnp.dot(x_ref[...], wr_ref[0],
                       preferred_element_type=jnp.float32)
        if relu:
            out = jnp.maximum(out, 0.0)
        o_ref[...] = out

    wsel = lambda j: jnp.maximum(j - (nblk - 3), 0)
    return pl.pallas_call(
        body,
        grid=(nblk,),
        in_specs=[
            pl.BlockSpec((2, 1024, HW), lambda j: (0, j, 0)),
            pl.BlockSpec((1024, D), lambda j: (j, 0)),
            pl.BlockSpec((1, D, D), lambda j: (wsel(j), 0, 0)),
            pl.BlockSpec((1, D, D), lambda j: (wsel(j), 0, 0)),
            pl.BlockSpec((1, 1, D), lambda j: (wsel(j), 0, 0)),
        ],
        out_specs=pl.BlockSpec((1024, D), lambda j: (j, 0)),
        out_shape=jax.ShapeDtypeStruct((NROWS, D), jnp.float32),
        interpret=interpret,
    )(sums, xs, Wl3, Wr3, bl3)


def _fixup(v, pop_row, ea_k, P, Wa_top, Wa_bot, ba_row, interpret=False):
    B = pop_row.shape[1]
    nj = B // 512

    def body(v_ref, pop_ref, ea_ref, p_ref, wt_ref, wb_ref, ba_ref,
             o_ref, last_s, acc_s):
        s = pl.program_id(0)

        @pl.when(s < nj)
        def _():
            popc = pop_ref[...]
            rowp = lax.broadcasted_iota(jnp.int32, (1024, 1), 0)
            eq = popc == rowp
            bi = lax.broadcasted_iota(jnp.int32, (1024, 512), 1) + s * 512
            chunk = jnp.max(jnp.where(eq, bi, -1), axis=1, keepdims=True)
            prev = jnp.where(s == 0, jnp.full((1024, 1), -1, jnp.int32),
                             last_s[...])
            last_s[...] = jnp.maximum(prev, chunk)

        @pl.when(s >= nj)
        def _():
            j = s - nj
            Bm = jnp.dot(p_ref[...], wb_ref[...],
                         preferred_element_type=jnp.float32)
            eaB = jnp.dot(ea_ref[...], Bm,
                          preferred_element_type=jnp.float32)
            bi = lax.broadcasted_iota(jnp.int32, (1024, 512), 1) + j * 512
            sel = (last_s[...] == bi).astype(jnp.float32)
            contrib = jnp.dot(sel, eaB, preferred_element_type=jnp.float32)
            acc = jnp.where(j == 0, jnp.zeros_like(contrib),
                            acc_s[...]) + contrib
            acc_s[...] = acc

            @pl.when(s == 2 * nj - 1)
            def _():
                A = jnp.dot(p_ref[...], wt_ref[...],
                            preferred_element_type=jnp.float32)
                vv = v_ref[...]
                upd = (jnp.dot(vv, A, preferred_element_type=jnp.float32)
                       + acc + ba_ref[0])
                o_ref[...] = jnp.where(last_s[...] >= 0, upd, vv)

    return pl.pallas_call(
        body,
        grid=(2 * nj,),
        in_specs=[
            pl.BlockSpec((1024, D), lambda s: (0, 0)),
            pl.BlockSpec((1, 512), lambda s: (0, lax.rem(s, nj))),
            pl.BlockSpec((512, D), lambda s: (jnp.maximum(s - nj, 0), 0)),
            pl.BlockSpec((D, D), lambda s: (0, 0)),
            pl.BlockSpec((D, D), lambda s: (0, 0)),
            pl.BlockSpec((D, D), lambda s: (0, 0)),
            pl.BlockSpec((1, D), lambda s: (0, 0)),
        ],
        out_specs=pl.BlockSpec((1024, D), lambda s: (0, 0)),
        out_shape=jax.ShapeDtypeStruct((1024, D), jnp.float32),
        scratch_shapes=[pltpu.VMEM((1024, 1), jnp.int32),
                        pltpu.VMEM((1024, D), jnp.float32)],
        interpret=interpret,
    )(v, pop_row, ea_k, P, Wa_top, Wa_bot, ba_row)


def _halves(x_stack):
    th = jnp.zeros((NC, NROWS, HW), jnp.float32)
    th = th.at[0, :, :64].set(x_stack[:, :64])
    th = th.at[1, :, :64].set(x_stack[:, 64:])
    th = th.at[:, :, 64].set(1.0)
    return th


def _run(x_individuals, x_occupation, x_residence, edge_index_family,
         edge_index_occupation, edge_index_residence, population,
         edge_attributes, Wl_dir_occ, bl_dir_occ, Wr_dir_occ, Wl_dir_res,
         bl_dir_res, Wr_dir_res, Wl_msg, bl_msg, Wr_msg, Wl_inv_occ,
         bl_inv_occ, Wr_inv_occ, Wl_inv_res, bl_inv_res, Wr_inv_res,
         Wl_inv_ind, bl_inv_ind, Wr_inv_ind, P_occ, P_res, W_aggr, b_aggr,
         sc_interpret=False, tc_interpret=False):
    f32, i32 = jnp.float32, jnp.int32
    ei_fam = edge_index_family.astype(i32)
    ei_occ = edge_index_occupation.astype(i32)
    ei_res = edge_index_residence.astype(i32)

    npad = E_PAD - E_TOT
    src1 = jnp.concatenate([ei_fam[1], ei_occ[0], ei_res[0],
                            jnp.zeros((npad,), i32)])
    dst_att = jnp.concatenate([ei_occ[1] + OFF_OCC, ei_res[1] + OFF_RES,
                               jnp.full((npad,), DUMMY, i32)])
    dst1 = jnp.concatenate([ei_fam[0], dst_att])
    src2 = jnp.concatenate([ei_fam[0], ei_occ[0], ei_res[0],
                            jnp.zeros((npad,), i32)])
    dst2 = jnp.concatenate([ei_fam[1], dst_att])
    srcs1 = src1.reshape(NS, NCHUNK, CH)
    dsts1 = dst1.reshape(NS, NCHUNK, CH)
    srcs2 = src2.reshape(NS, NCHUNK, CH)
    dsts2 = dst2.reshape(NS, NCHUNK, CH)
    zeros96 = jnp.zeros((96, HW), f32)

    xs = jnp.zeros((NROWS, D), f32)
    xs = lax.dynamic_update_slice(xs, x_individuals, (0, 0))
    xs = lax.dynamic_update_slice(xs, x_occupation, (OFF_OCC, 0))
    xs = lax.dynamic_update_slice(xs, x_residence, (OFF_RES, 0))

    Wl3a = jnp.stack([Wl_msg, Wl_dir_occ, Wl_dir_res])
    Wr3a = jnp.stack([Wr_msg, Wr_dir_occ, Wr_dir_res])
    bl3a = jnp.stack([bl_msg, bl_dir_occ, bl_dir_res])[:, None, :]
    Wl3b = jnp.stack([Wl_inv_ind, Wl_inv_occ, Wl_inv_res])
    Wr3b = jnp.stack([Wr_inv_ind, Wr_inv_occ, Wr_inv_res])
    bl3b = jnp.stack([bl_inv_ind, bl_inv_occ, bl_inv_res])[:, None, :]

    sums1 = _segsum_sc(_halves(xs), srcs1, dsts1, zeros96,
                       interpret=sc_interpret)
    x2 = _dense_layer(sums1, xs, Wl3a, Wr3a, bl3a, relu=True,
                      interpret=tc_interpret)

    sums2 = _segsum_sc(_halves(x2), srcs2, dsts2, zeros96,
                       interpret=sc_interpret)
    y2 = _dense_layer(sums2, x2, Wl3b, Wr3b, bl3b, relu=False,
                      interpret=tc_interpret)

    pop_row = population.astype(i32).reshape(1, -1)
    Wa_top, Wa_bot = W_aggr[:D], W_aggr[D:]
    ba_row = b_aggr.reshape(1, D)
    occ2 = _fixup(y2[OFF_OCC:OFF_OCC + 1024], pop_row,
                  edge_attributes[:, 0, :], P_occ, Wa_top, Wa_bot, ba_row,
                  interpret=tc_interpret)
    res2 = _fixup(y2[OFF_RES:OFF_RES + 1024], pop_row,
                  edge_attributes[:, 1, :], P_res, Wa_top, Wa_bot, ba_row,
                  interpret=tc_interpret)

    return (y2[:N_IND], occ2[:N_ATT], res2[:N_ATT])


def kernel(x_individuals, x_occupation, x_residence, edge_index_family,
           edge_index_occupation, edge_index_residence, population,
           edge_attributes, Wl_dir_occ, bl_dir_occ, Wr_dir_occ, Wl_dir_res,
           bl_dir_res, Wr_dir_res, Wl_msg, bl_msg, Wr_msg, Wl_inv_occ,
           bl_inv_occ, Wr_inv_occ, Wl_inv_res, bl_inv_res, Wr_inv_res,
           Wl_inv_ind, bl_inv_ind, Wr_inv_ind, P_occ, P_res, W_aggr, b_aggr):
    return _run(x_individuals, x_occupation, x_residence, edge_index_family,
                edge_index_occupation, edge_index_residence, population,
                edge_attributes, Wl_dir_occ, bl_dir_occ, Wr_dir_occ,
                Wl_dir_res, bl_dir_res, Wr_dir_res, Wl_msg, bl_msg, Wr_msg,
                Wl_inv_occ, bl_inv_occ, Wr_inv_occ, Wl_inv_res, bl_inv_res,
                Wr_inv_res, Wl_inv_ind, bl_inv_ind, Wr_inv_ind, P_occ, P_res,
                W_aggr, b_aggr)
